# Initial kernel scaffold; baseline (speedup 1.0000x reference)
import jax
import jax.numpy as jnp
from jax.experimental import pallas as pl


def kernel(x, edge_index, edge_weight, ent_emb, rel_trans):
    raise NotImplementedError("write your pallas kernel here")


# speedup vs baseline: 6.3695x; 6.3695x over previous
import functools

import jax
import jax.numpy as jnp
from jax import lax
from jax.experimental import pallas as pl
from jax.experimental.pallas import tpu as pltpu
from jax.experimental.pallas import tpu_sc as plsc

N = 10000
R = 4
E = 150000
D = 128

NC = 2
NS = 16
NW = NC * NS
LANES = 16

C = 240
CHUNKS_PER_REL = E // C
TOTAL_CHUNKS = R * CHUNKS_PER_REL
ITERS = (TOTAL_CHUNKS + NW - 1) // NW
ROW_STRIDE = 624
ROW_SPAN = N - ROW_STRIDE * (NS - 1)

_sc_mesh = plsc.VectorSubcoreMesh(core_axis_name="c", subcore_axis_name="s")


@functools.partial(
    pl.kernel,
    out_type=jax.ShapeDtypeStruct((NC * N, D), jnp.float32),
    mesh=_sc_mesh,
    scratch_types=[
        pltpu.VMEM((C,), jnp.int32),
        pltpu.VMEM((C,), jnp.int32),
        pltpu.VMEM((C,), jnp.float32),
        pltpu.VMEM((C, D), jnp.float32),
        pltpu.VMEM_SHARED((N, D), jnp.float32),
        pltpu.SemaphoreType.DMA,
    ],
)
def _sc_aggregate(table, srcs, dsts, ws, out, idx_v, dst_v, w_v, rows_v,
                  agg, sem):
    c = lax.axis_index("c")
    s = lax.axis_index("s")
    wid = s * NC + c

    def _zero_row(e, carry):
        for j in range(D // LANES):
            rows_v[e, pl.ds(j * LANES, LANES)] = jnp.zeros((LANES,),
                                                           jnp.float32)
        return carry

    lax.fori_loop(0, C, _zero_row, 0)
    base = s * ROW_STRIDE
    off = 0
    while off < ROW_SPAN:
        size = min(C, ROW_SPAN - off)
        pltpu.sync_copy(rows_v.at[pl.ds(0, size)],
                        agg.at[pl.ds(base + off, size)])
        off += size
    plsc.subcore_barrier()

    def _chunk(it, carry):
        k = wid + it * NW

        @pl.when(k < TOTAL_CHUNKS)
        def _():
            r = k // CHUNKS_PER_REL
            off = k * C
            pltpu.sync_copy(srcs.at[pl.ds(off, C)], idx_v)
            pltpu.sync_copy(dsts.at[pl.ds(off, C)], dst_v)
            pltpu.sync_copy(ws.at[pl.ds(off, C)], w_v)
            shift = r * N
            for i in range(C // LANES):
                sl = pl.ds(i * LANES, LANES)
                idx_v[sl] = idx_v[sl] + shift
            pltpu.async_copy(table.at[idx_v], rows_v, sem).wait()
            def _scale(g, carry2):
                wvec = w_v[pl.ds(g * LANES, LANES)]
                for e16 in range(LANES):
                    row = g * LANES + e16
                    wv = wvec[e16]
                    for j in range(D // LANES):
                        sl = pl.ds(j * LANES, LANES)
                        rows_v[row, sl] = rows_v[row, sl] * wv
                return carry2

            lax.fori_loop(0, C // LANES, _scale, 0)
            pltpu.sync_copy(rows_v, agg.at[dst_v], add=True)

        return carry

    lax.fori_loop(0, ITERS, _chunk, 0)
    plsc.subcore_barrier()
    pltpu.sync_copy(agg.at[pl.ds(base, ROW_SPAN)],
                    out.at[pl.ds(c * N + base, ROW_SPAN)])


BM = 1000


def _first_tables_body(x_ref, e_ref, w_ref, o_ref):
    emb = jnp.dot(x_ref[...], e_ref[...], preferred_element_type=jnp.float32)
    for r in range(R):
        o_ref[r] = lax.dot_general(emb, w_ref[r], (((1,), (1,)), ((), ())),
                                   preferred_element_type=jnp.float32)


def _first_tables(x, ent_emb, w):
    return pl.pallas_call(
        _first_tables_body,
        grid=(N // BM,),
        in_specs=[
            pl.BlockSpec((BM, D), lambda i: (i, 0)),
            pl.BlockSpec((D, D), lambda i: (0, 0)),
            pl.BlockSpec((R, D, D), lambda i: (0, 0, 0)),
        ],
        out_specs=pl.BlockSpec((R, BM, D), lambda i: (0, i, 0)),
        out_shape=jax.ShapeDtypeStruct((R, N, D), jnp.float32),
    )(x, ent_emb, w)


def _mid_tables_body(p_ref, w_ref, o_ref):
    emb = jax.nn.relu(p_ref[0] + p_ref[1])
    for r in range(R):
        o_ref[r] = lax.dot_general(emb, w_ref[r], (((1,), (1,)), ((), ())),
                                   preferred_element_type=jnp.float32)


def _mid_tables(p, w):
    return pl.pallas_call(
        _mid_tables_body,
        grid=(N // BM,),
        in_specs=[
            pl.BlockSpec((NC, BM, D), lambda i: (0, i, 0)),
            pl.BlockSpec((R, D, D), lambda i: (0, 0, 0)),
        ],
        out_specs=pl.BlockSpec((R, BM, D), lambda i: (0, i, 0)),
        out_shape=jax.ShapeDtypeStruct((R, N, D), jnp.float32),
    )(p, w)


def _finalize_body(p_ref, o_ref):
    emb = jax.nn.relu(p_ref[0] + p_ref[1])
    nrm = jnp.sqrt(jnp.sum(emb * emb, axis=1, keepdims=True))
    o_ref[...] = emb / jnp.maximum(nrm, 1e-12)


def _finalize(p):
    return pl.pallas_call(
        _finalize_body,
        grid=(N // BM,),
        in_specs=[pl.BlockSpec((NC, BM, D), lambda i: (0, i, 0))],
        out_specs=pl.BlockSpec((BM, D), lambda i: (i, 0)),
        out_shape=jax.ShapeDtypeStruct((N, D), jnp.float32),
    )(p)


def kernel(x, edge_index, edge_weight, ent_emb, rel_trans):
    srcs = edge_index[:, 1, :].reshape(-1)
    dsts = edge_index[:, 0, :].reshape(-1)
    ws = edge_weight.reshape(-1)
    n_layers = rel_trans.shape[0]
    tables = _first_tables(x, ent_emb, rel_trans[0])
    for l in range(n_layers):
        partial = _sc_aggregate(tables.reshape(R * N, D), srcs, dsts, ws)
        partial = partial.reshape(NC, N, D)
        if l + 1 < n_layers:
            tables = _mid_tables(partial, rel_trans[l + 1])
        else:
            return _finalize(partial)
